# baseline (speedup 1.0000x reference)
import functools

import jax
import jax.numpy as jnp
from jax.experimental import pallas as pl
from jax.experimental.pallas import tpu as pltpu

NUM_EMBED_K = 100000
EMB_DIM_K = 768
TOPK_K = 8
N_TOKENS_K = 64

BLOCK_ROWS = 2048
SUB = 512


def _topk_kernel(x_ref, w_ref, vals_ref, idx_ref, run_v_ref, run_i_ref):
    i = pl.program_id(0)
    nsteps = pl.num_programs(0)

    @pl.when(i == 0)
    def _init():
        run_v_ref[...] = jnp.full(run_v_ref.shape, -jnp.inf, jnp.float32)
        run_i_ref[...] = jnp.zeros(run_i_ref.shape, jnp.int32)

    x = x_ref[...]
    w = w_ref[...]
    logits = jax.lax.dot_general(
        x, w, (((1,), (1,)), ((), ())), preferred_element_type=jnp.float32
    )

    b = logits.shape[1]
    base = i * b
    cols = base + jax.lax.broadcasted_iota(jnp.int32, logits.shape, 1)
    logits = jnp.where(cols < NUM_EMBED_K, logits, -jnp.inf)

    BIG = jnp.int32(2**30)
    for s in range(b // SUB):
        sub_v = logits[:, s * SUB:(s + 1) * SUB]
        sub_i = cols[:, s * SUB:(s + 1) * SUB]
        run_min = jnp.min(run_v_ref[:, 0:TOPK_K], axis=1, keepdims=True)
        need = jnp.any(sub_v > run_min)

        @pl.when(need)
        def _merge(sub_v=sub_v, sub_i=sub_i):
            ext_v = jnp.concatenate([sub_v, run_v_ref[...]], axis=1)
            ext_i = jnp.concatenate([sub_i, run_i_ref[...]], axis=1)
            out_vs = []
            out_is = []
            for _ in range(TOPK_K):
                m = jnp.max(ext_v, axis=1, keepdims=True)
                gi = jnp.min(
                    jnp.where(ext_v == m, ext_i, BIG), axis=1, keepdims=True
                )
                out_vs.append(m)
                out_is.append(gi)
                ext_v = jnp.where(ext_i == gi, -jnp.inf, ext_v)

            run_v_ref[:, 0:TOPK_K] = jnp.concatenate(out_vs, axis=1)
            run_i_ref[:, 0:TOPK_K] = jnp.concatenate(out_is, axis=1)

    @pl.when(i == nsteps - 1)
    def _emit():
        vals_ref[...] = run_v_ref[:, 0:TOPK_K]
        idx_ref[...] = run_i_ref[:, 0:TOPK_K]


@jax.jit
def kernel(x, W):
    nsteps = pl.cdiv(NUM_EMBED_K, BLOCK_ROWS)
    vals, idx = pl.pallas_call(
        _topk_kernel,
        grid=(nsteps,),
        in_specs=[
            pl.BlockSpec((N_TOKENS_K, EMB_DIM_K), lambda i: (0, 0)),
            pl.BlockSpec((BLOCK_ROWS, EMB_DIM_K), lambda i: (i, 0)),
        ],
        out_specs=[
            pl.BlockSpec((N_TOKENS_K, TOPK_K), lambda i: (0, 0)),
            pl.BlockSpec((N_TOKENS_K, TOPK_K), lambda i: (0, 0)),
        ],
        out_shape=[
            jax.ShapeDtypeStruct((N_TOKENS_K, TOPK_K), jnp.float32),
            jax.ShapeDtypeStruct((N_TOKENS_K, TOPK_K), jnp.int32),
        ],
        scratch_shapes=[
            pltpu.VMEM((N_TOKENS_K, 128), jnp.float32),
            pltpu.VMEM((N_TOKENS_K, 128), jnp.int32),
        ],
    )(x, W)
    return (vals, idx)


# speedup vs baseline: 2.7847x; 2.7847x over previous
import jax
import jax.numpy as jnp
from jax.experimental import pallas as pl
from jax.experimental.pallas import tpu as pltpu

NUM_EMBED_K = 100000
EMB_DIM_K = 768
TOPK_K = 8
N_TOKENS_K = 64

BLOCK_ROWS = 2048
NBLOCKS = (NUM_EMBED_K + BLOCK_ROWS - 1) // BLOCK_ROWS
PAD_COLS = NBLOCKS * BLOCK_ROWS
TAIL_VALID = NUM_EMBED_K - (NBLOCKS - 1) * BLOCK_ROWS


def _topk_kernel(x_ref, w_ref, vals_ref, idx_ref, logit_ref):
    i = pl.program_id(0)
    nsteps = pl.num_programs(0)

    x = x_ref[...]
    w = w_ref[...]
    logits = jax.lax.dot_general(
        x, w, (((1,), (1,)), ((), ())), preferred_element_type=jnp.float32
    )
    logit_ref[i] = logits

    @pl.when(i == nsteps - 1)
    def _tail_mask():
        logit_ref[i, :, TAIL_VALID:] = jnp.full(
            (N_TOKENS_K, BLOCK_ROWS - TAIL_VALID), -jnp.inf, jnp.float32
        )

    @pl.when(i == nsteps - 1)
    def _extract():
        BIG = jnp.int32(2**30)
        iota = jax.lax.broadcasted_iota(
            jnp.int32, (N_TOKENS_K, BLOCK_ROWS), 1
        )
        m_out = []
        g_out = []
        gi_prev = jnp.full((N_TOKENS_K, 1), -1, jnp.int32)
        for _ in range(TOPK_K):
            gp = gi_prev

            def body(c, carry, gp=gp):
                M, I = carry
                v = logit_ref[c]
                idx = iota + c * BLOCK_ROWS
                v = jnp.where(idx == gp, -jnp.inf, v)
                logit_ref[c] = v
                for t in range(BLOCK_ROWS // 128):
                    sv = v[:, t * 128:(t + 1) * 128]
                    si = idx[:, t * 128:(t + 1) * 128]
                    upd = (sv > M) | ((sv == M) & (si < I))
                    M = jnp.where(upd, sv, M)
                    I = jnp.where(upd, si, I)
                return M, I

            M0 = jnp.full((N_TOKENS_K, 128), -jnp.inf, jnp.float32)
            I0 = jnp.full((N_TOKENS_K, 128), BIG, jnp.int32)
            M, I = jax.lax.fori_loop(0, nsteps, body, (M0, I0))
            m = jnp.max(M, axis=1, keepdims=True)
            gi = jnp.min(jnp.where(M == m, I, BIG), axis=1, keepdims=True)
            m_out.append(m)
            g_out.append(gi)
            gi_prev = gi

        vals_ref[...] = jnp.concatenate(m_out, axis=1)
        idx_ref[...] = jnp.concatenate(g_out, axis=1)


@jax.jit
def kernel(x, W):
    vals, idx = pl.pallas_call(
        _topk_kernel,
        grid=(NBLOCKS,),
        in_specs=[
            pl.BlockSpec((N_TOKENS_K, EMB_DIM_K), lambda i: (0, 0)),
            pl.BlockSpec((BLOCK_ROWS, EMB_DIM_K), lambda i: (i, 0)),
        ],
        out_specs=[
            pl.BlockSpec((N_TOKENS_K, TOPK_K), lambda i: (0, 0)),
            pl.BlockSpec((N_TOKENS_K, TOPK_K), lambda i: (0, 0)),
        ],
        out_shape=[
            jax.ShapeDtypeStruct((N_TOKENS_K, TOPK_K), jnp.float32),
            jax.ShapeDtypeStruct((N_TOKENS_K, TOPK_K), jnp.int32),
        ],
        scratch_shapes=[
            pltpu.VMEM((NBLOCKS, N_TOKENS_K, BLOCK_ROWS), jnp.float32),
        ],
    )(x, W)
    return (vals, idx)


# speedup vs baseline: 2.9390x; 1.0554x over previous
import jax
import jax.numpy as jnp
from jax.experimental import pallas as pl
from jax.experimental.pallas import tpu as pltpu

NUM_EMBED_K = 100000
EMB_DIM_K = 768
TOPK_K = 8
N_TOKENS_K = 64

BLOCK_ROWS = 2048
NBLOCKS = (NUM_EMBED_K + BLOCK_ROWS - 1) // BLOCK_ROWS
PAD_COLS = NBLOCKS * BLOCK_ROWS
TAIL_VALID = NUM_EMBED_K - (NBLOCKS - 1) * BLOCK_ROWS


GRID_STEPS = 25


def _topk_kernel(x_ref, wa_ref, wb_ref, vals_ref, idx_ref, logit_ref):
    i = pl.program_id(0)
    nsteps = pl.num_programs(0)

    x = x_ref[...]
    la = jax.lax.dot_general(
        x, wa_ref[...], (((1,), (1,)), ((), ())),
        preferred_element_type=jnp.float32,
    )
    logit_ref[i] = la

    @pl.when(i < GRID_STEPS - 1)
    def _do_b():
        lb = jax.lax.dot_general(
            x, wb_ref[...], (((1,), (1,)), ((), ())),
            preferred_element_type=jnp.float32,
        )
        logit_ref[GRID_STEPS + i] = lb

    @pl.when(i == nsteps - 1)
    def _tail_mask():
        logit_ref[NBLOCKS - 1, :, TAIL_VALID:] = jnp.full(
            (N_TOKENS_K, BLOCK_ROWS - TAIL_VALID), -jnp.inf, jnp.float32
        )

    @pl.when(i == nsteps - 1)
    def _extract():
        BIG = jnp.int32(2**30)
        iota = jax.lax.broadcasted_iota(
            jnp.int32, (N_TOKENS_K, BLOCK_ROWS), 1
        )
        m_out = []
        g_out = []
        gi_prev = jnp.full((N_TOKENS_K, 1), -1, jnp.int32)
        for _ in range(TOPK_K):
            gp = gi_prev

            def body(c, carry, gp=gp):
                M, I = carry
                v = logit_ref[c]
                idx = iota + c * BLOCK_ROWS
                v = jnp.where(idx == gp, -jnp.inf, v)
                logit_ref[c] = v
                for t in range(BLOCK_ROWS // 128):
                    sv = v[:, t * 128:(t + 1) * 128]
                    si = idx[:, t * 128:(t + 1) * 128]
                    upd = (sv > M) | ((sv == M) & (si < I))
                    M = jnp.where(upd, sv, M)
                    I = jnp.where(upd, si, I)
                return M, I

            M0 = jnp.full((N_TOKENS_K, 128), -jnp.inf, jnp.float32)
            I0 = jnp.full((N_TOKENS_K, 128), BIG, jnp.int32)
            M, I = jax.lax.fori_loop(0, NBLOCKS, body, (M0, I0))
            m = jnp.max(M, axis=1, keepdims=True)
            gi = jnp.min(jnp.where(M == m, I, BIG), axis=1, keepdims=True)
            m_out.append(m)
            g_out.append(gi)
            gi_prev = gi

        vals_ref[...] = jnp.concatenate(m_out, axis=1)
        idx_ref[...] = jnp.concatenate(g_out, axis=1)


@jax.jit
def kernel(x, W):
    vals, idx = pl.pallas_call(
        _topk_kernel,
        grid=(GRID_STEPS,),
        in_specs=[
            pl.BlockSpec((N_TOKENS_K, EMB_DIM_K), lambda i: (0, 0)),
            pl.BlockSpec((BLOCK_ROWS, EMB_DIM_K), lambda i: (i, 0)),
            pl.BlockSpec(
                (BLOCK_ROWS, EMB_DIM_K),
                lambda i: (GRID_STEPS + jnp.minimum(i, GRID_STEPS - 2), 0),
            ),
        ],
        out_specs=[
            pl.BlockSpec((N_TOKENS_K, TOPK_K), lambda i: (0, 0)),
            pl.BlockSpec((N_TOKENS_K, TOPK_K), lambda i: (0, 0)),
        ],
        out_shape=[
            jax.ShapeDtypeStruct((N_TOKENS_K, TOPK_K), jnp.float32),
            jax.ShapeDtypeStruct((N_TOKENS_K, TOPK_K), jnp.int32),
        ],
        scratch_shapes=[
            pltpu.VMEM((NBLOCKS, N_TOKENS_K, BLOCK_ROWS), jnp.float32),
        ],
    )(x, W, W)
    return (vals, idx)
